# baseline (speedup 1.0000x reference)
import functools
import math

import jax
import jax.numpy as jnp
from jax.experimental import pallas as pl

_BLOCK = 256


def _indexed_linear_kernel(x_ref, t_ref, w_ref, o_ref, *, n_types, alpha):
    xb = x_ref[...]
    tb = t_ref[0]
    acc = jnp.zeros((xb.shape[0], w_ref.shape[2]), jnp.float32)
    for t in range(n_types):
        mask = tb == t
        xm = jnp.where(mask, xb, 0.0)
        acc = acc + jnp.dot(xm, w_ref[t], preferred_element_type=jnp.float32)
    o_ref[...] = acc * alpha


def kernel(x, node_types, W):
    n, d_in = x.shape
    n_types, _, d_out = W.shape
    alpha = 1.0 / math.sqrt(d_in)

    n_pad = ((n + _BLOCK - 1) // _BLOCK) * _BLOCK
    grid = n_pad // _BLOCK
    if n_pad != n:
        x = jnp.pad(x, ((0, n_pad - n), (0, 0)))
        node_types = jnp.pad(node_types, (0, n_pad - n))
    t3 = node_types.astype(jnp.int32).reshape(grid, _BLOCK, 1)

    out = pl.pallas_call(
        functools.partial(_indexed_linear_kernel, n_types=n_types, alpha=alpha),
        grid=(grid,),
        in_specs=[
            pl.BlockSpec((_BLOCK, d_in), lambda i: (i, 0)),
            pl.BlockSpec((1, _BLOCK, 1), lambda i: (i, 0, 0)),
            pl.BlockSpec((n_types, d_in, d_out), lambda i: (0, 0, 0)),
        ],
        out_specs=pl.BlockSpec((_BLOCK, d_out), lambda i: (i, 0)),
        out_shape=jax.ShapeDtypeStruct((n_pad, d_out), jnp.float32),
    )(x, t3, W)
    return out[:n]
